# baseline (speedup 1.0000x reference)
import jax
import jax.numpy as jnp
from jax import lax
from jax.experimental import pallas as pl
from jax.experimental.pallas import tpu as pltpu

_E = 64
_T = 32768
_BT = 2048
_G = _T // _BT
_LANES = 128


def _body(x_ref, nt_ref, o_ref, accv):
    i = pl.program_id(0)

    @pl.when(i == 0)
    def _():
        accv[...] = jnp.zeros_like(accv)

    ones = jnp.ones((1, _E), jnp.float32)
    acc = accv[...]
    for j in range(_BT // _LANES):
        ej = jnp.exp(x_ref[:, j * _LANES:(j + 1) * _LANES])
        dj = lax.dot_general(ones, ej, (((1,), (0,)), ((), ())),
                             preferred_element_type=jnp.float32)
        acc = acc + ej * (1.0 / dj)
    accv[...] = acc

    @pl.when(i == _G - 1)
    def _():
        spe = jnp.sum(accv[...], axis=1, keepdims=True)
        ntf = nt_ref[...].astype(jnp.float32)
        nts = lax.dot_general(ntf, spe, (((1,), (0,)), ((), ())))[0, 0]
        sum_nt = jnp.sum(ntf)
        balance = (_E / _T) * nts / sum_nt
        sum_s = jnp.sum(spe)
        sum_s2 = jnp.sum(spe * spe)
        m = sum_s / _E
        var = (sum_s2 - _E * m * m) / (_E - 1)
        o_ref[...] = (balance + var / (m * m)).reshape(1, 1)


def kernel(router_logits, num_tokens):
    out = pl.pallas_call(
        _body,
        grid=(_G,),
        in_specs=[
            pl.BlockSpec((_E, _BT), lambda i: (0, i)),
            pl.BlockSpec((1, _E), lambda i: (0, 0)),
        ],
        out_specs=pl.BlockSpec((1, 1), lambda i: (0, 0)),
        out_shape=jax.ShapeDtypeStruct((1, 1), jnp.float32),
        compiler_params=pltpu.CompilerParams(vmem_limit_bytes=6 * 1024 * 1024),
        scratch_shapes=[pltpu.VMEM((_E, _LANES), jnp.float32)],
    )(pltpu.with_memory_space_constraint(router_logits.T, pltpu.MemorySpace.HBM),
      num_tokens.reshape(1, _E))
    return out[0, 0]


# speedup vs baseline: 2.3642x; 2.3642x over previous
import jax
import jax.numpy as jnp
from jax import lax
from jax.experimental import pallas as pl
from jax.experimental.pallas import tpu as pltpu

_E = 64
_T = 32768
_BT = 16384
_G = _T // _BT
_LANES = 128


def _body(x_ref, nt_ref, o_ref, accv):
    i = pl.program_id(0)

    @pl.when(i == 0)
    def _():
        accv[...] = jnp.zeros_like(accv)

    ones = jnp.ones((1, _E), jnp.float32)
    acc = accv[...]
    for j in range(_BT // _LANES):
        ej = jnp.exp(x_ref[:, j * _LANES:(j + 1) * _LANES])
        dj = lax.dot_general(ones, ej, (((1,), (0,)), ((), ())),
                             preferred_element_type=jnp.float32)
        acc = acc + ej * (1.0 / dj)
    accv[...] = acc

    @pl.when(i == _G - 1)
    def _():
        spe = jnp.sum(accv[...], axis=1, keepdims=True)
        ntf = nt_ref[...].astype(jnp.float32)
        nts = lax.dot_general(ntf, spe, (((1,), (0,)), ((), ())))[0, 0]
        sum_nt = jnp.sum(ntf)
        balance = (_E / _T) * nts / sum_nt
        sum_s = jnp.sum(spe)
        sum_s2 = jnp.sum(spe * spe)
        m = sum_s / _E
        var = (sum_s2 - _E * m * m) / (_E - 1)
        o_ref[...] = (balance + var / (m * m)).reshape(1, 1)


def kernel(router_logits, num_tokens):
    out = pl.pallas_call(
        _body,
        grid=(_G,),
        in_specs=[
            pl.BlockSpec((_E, _BT), lambda i: (0, i)),
            pl.BlockSpec((1, _E), lambda i: (0, 0)),
        ],
        out_specs=pl.BlockSpec((1, 1), lambda i: (0, 0)),
        out_shape=jax.ShapeDtypeStruct((1, 1), jnp.float32),
        scratch_shapes=[pltpu.VMEM((_E, _LANES), jnp.float32)],
    )(router_logits.T, num_tokens.reshape(1, _E))
    return out[0, 0]
